# baseline (speedup 1.0000x reference)
import functools

import jax
import jax.numpy as jnp
from jax import lax
from jax.experimental import pallas as pl
from jax.experimental.pallas import tpu as pltpu
from jax.experimental.pallas import tpu_sc as plsc

_B = 16384
_D = 16

_info = plsc.get_sparse_core_info()
_NC = _info.num_cores
_NS = _info.num_subcores
_NW = _NC * _NS
_BPW = _B // _NW

_mesh = plsc.VectorSubcoreMesh(core_axis_name="c", subcore_axis_name="s")


@functools.partial(
    pl.kernel,
    mesh=_mesh,
    compiler_params=pltpu.CompilerParams(use_tc_tiling_on_sc=True),
    out_type=(
        jax.ShapeDtypeStruct((_B, _D), jnp.float32),
        jax.ShapeDtypeStruct((_B, _D), jnp.float32),
    ),
    scratch_types=[
        pltpu.VMEM((_BPW,), jnp.int32),
        pltpu.VMEM((_BPW,), jnp.int32),
        pltpu.SemaphoreType.DMA,
        pltpu.SemaphoreType.DMA,
    ],
)
def _dual_gather(ctab, rtab, cids, rids, xr, xc,
                 cidx_v, ridx_v, sem_c, sem_r):
    wid = lax.axis_index("s") * _NC + lax.axis_index("c")
    base = wid * _BPW
    pltpu.sync_copy(rids.at[pl.ds(base, _BPW)], ridx_v)
    pltpu.sync_copy(cids.at[pl.ds(base, _BPW)], cidx_v)

    def body(g, carry):
        off = g * 16
        rvec = ridx_v[pl.ds(off, 16)]
        cvec = cidx_v[pl.ds(off, 16)]
        for j in range(16):
            pltpu.make_async_copy(
                rtab.at[pl.ds(rvec[j], 1), :],
                xr.at[pl.ds(base + off + j, 1), :],
                sem_r,
            ).start()
            pltpu.make_async_copy(
                ctab.at[pl.ds(cvec[j], 1), :],
                xc.at[pl.ds(base + off + j, 1), :],
                sem_c,
            ).start()
        return carry

    lax.fori_loop(0, _BPW // 16, body, 0)
    pltpu.make_async_copy(
        rtab.at[pl.ds(0, _BPW), :], xr.at[pl.ds(base, _BPW)], sem_r
    ).wait()
    pltpu.make_async_copy(
        ctab.at[pl.ds(0, _BPW), :], xc.at[pl.ds(base, _BPW)], sem_c
    ).wait()


def kernel(contributor_table, recipient_table, contributor_ids, recipient_ids):
    xr, xc = _dual_gather(
        contributor_table,
        recipient_table,
        contributor_ids.astype(jnp.int32),
        recipient_ids.astype(jnp.int32),
    )
    return (xr, xc)


# speedup vs baseline: 9.8766x; 9.8766x over previous
import functools

import jax
import jax.numpy as jnp
from jax import lax
from jax.experimental import pallas as pl
from jax.experimental.pallas import tpu as pltpu
from jax.experimental.pallas import tpu_sc as plsc

_B = 16384
_D = 16
_V = 100000

_info = plsc.get_sparse_core_info()
_NC = _info.num_cores
_NS = _info.num_subcores
_NW = _NC * _NS
_BPW = _B // _NW
_NT = _BPW // 128

_mesh = plsc.VectorSubcoreMesh(core_axis_name="c", subcore_axis_name="s")


def _gather_panel(tab, ids, out, idv, idxb0, idxb1, rbuf0, rbuf1, sem, base):
    pltpu.sync_copy(ids.at[pl.ds(base, _BPW)], idv)
    for o in range(_NT):
        for s in range(8):
            vec = idv[pl.ds(o * 128 + s * 16, 16)]
            for dd in range(8):
                idxb0[o, dd, pl.ds(s * 16, 16)] = vec + (dd * _V)
                idxb1[o, dd, pl.ds(s * 16, 16)] = vec + ((8 + dd) * _V)
    copies = []
    for o in range(_NT):
        for dd in range(8):
            copies.append(pltpu.async_copy(
                tab.at[idxb0.at[o, dd]],
                rbuf0.at[dd, pl.ds(o * 128, 128)], sem))
            copies.append(pltpu.async_copy(
                tab.at[idxb1.at[o, dd]],
                rbuf1.at[dd, pl.ds(o * 128, 128)], sem))
    for cp in copies:
        cp.wait()
    pltpu.sync_copy(rbuf0, out.at[pl.ds(0, 8), pl.ds(base, _BPW)])
    pltpu.sync_copy(rbuf1, out.at[pl.ds(8, 8), pl.ds(base, _BPW)])


@functools.partial(
    pl.kernel,
    mesh=_mesh,
    compiler_params=pltpu.CompilerParams(use_tc_tiling_on_sc=True),
    out_type=(
        jax.ShapeDtypeStruct((_D, _B), jnp.float32),
        jax.ShapeDtypeStruct((_D, _B), jnp.float32),
    ),
    scratch_types=[
        pltpu.VMEM((_BPW,), jnp.int32),
        pltpu.VMEM((_NT, 8, 128), jnp.int32),
        pltpu.VMEM((_NT, 8, 128), jnp.int32),
        pltpu.VMEM((8, _BPW), jnp.float32),
        pltpu.VMEM((8, _BPW), jnp.float32),
        pltpu.SemaphoreType.DMA,
    ],
)
def _dual_gather(ctab, rtab, cids, rids, xrT, xcT,
                 idv, idxb0, idxb1, rbuf0, rbuf1, sem):
    wid = lax.axis_index("s") * _NC + lax.axis_index("c")
    base = wid * _BPW
    _gather_panel(rtab, rids, xrT, idv, idxb0, idxb1, rbuf0, rbuf1, sem, base)
    _gather_panel(ctab, cids, xcT, idv, idxb0, idxb1, rbuf0, rbuf1, sem, base)


def kernel(contributor_table, recipient_table, contributor_ids, recipient_ids):
    cflat = contributor_table.T.reshape(-1)
    rflat = recipient_table.T.reshape(-1)
    xrT, xcT = _dual_gather(
        cflat,
        rflat,
        contributor_ids.astype(jnp.int32),
        recipient_ids.astype(jnp.int32),
    )
    return (xrT.T, xcT.T)


# speedup vs baseline: 10.2114x; 1.0339x over previous
import functools

import jax
import jax.numpy as jnp
from jax import lax
from jax.experimental import pallas as pl
from jax.experimental.pallas import tpu as pltpu
from jax.experimental.pallas import tpu_sc as plsc

_B = 16384
_D = 16
_V = 100000

_info = plsc.get_sparse_core_info()
_NC = _info.num_cores
_NS = _info.num_subcores
_NW = _NC * _NS
_BPW = _B // _NW
_NT = _BPW // 128

_mesh = plsc.VectorSubcoreMesh(core_axis_name="c", subcore_axis_name="s")


@functools.partial(
    pl.kernel,
    mesh=_mesh,
    compiler_params=pltpu.CompilerParams(use_tc_tiling_on_sc=True),
    out_type=jax.ShapeDtypeStruct((_D, _B), jnp.float32),
    scratch_types=[
        pltpu.VMEM((_BPW,), jnp.int32),
        pltpu.VMEM((_NT, 8, 128), jnp.int32),
        pltpu.VMEM((_NT, 8, 128), jnp.int32),
        pltpu.VMEM((8, _BPW), jnp.float32),
        pltpu.VMEM((8, _BPW), jnp.float32),
        pltpu.SemaphoreType.DMA,
    ],
)
def _gather_one(tab, ids, out, idv, idxb0, idxb1, rbuf0, rbuf1, sem):
    wid = lax.axis_index("s") * _NC + lax.axis_index("c")
    base = wid * _BPW
    pltpu.sync_copy(ids.at[pl.ds(base, _BPW)], idv)
    copies = []
    for o in range(_NT):
        for s in range(8):
            vec = idv[pl.ds(o * 128 + s * 16, 16)]
            for dd in range(8):
                idxb0[o, dd, pl.ds(s * 16, 16)] = vec + (dd * _V)
                idxb1[o, dd, pl.ds(s * 16, 16)] = vec + ((8 + dd) * _V)
        for dd in range(8):
            copies.append(pltpu.async_copy(
                tab.at[idxb0.at[o, dd]],
                rbuf0.at[dd, pl.ds(o * 128, 128)], sem))
            copies.append(pltpu.async_copy(
                tab.at[idxb1.at[o, dd]],
                rbuf1.at[dd, pl.ds(o * 128, 128)], sem))
    for cp in copies:
        cp.wait()
    pltpu.sync_copy(rbuf0, out.at[pl.ds(0, 8), pl.ds(base, _BPW)])
    pltpu.sync_copy(rbuf1, out.at[pl.ds(8, 8), pl.ds(base, _BPW)])


def kernel(contributor_table, recipient_table, contributor_ids, recipient_ids):
    rflat = recipient_table.T.reshape(-1)
    cflat = contributor_table.T.reshape(-1)
    xrT = _gather_one(rflat, recipient_ids.astype(jnp.int32))
    xcT = _gather_one(cflat, contributor_ids.astype(jnp.int32))
    return (xrT.T, xcT.T)
